# baseline (speedup 1.0000x reference)
import jax
import jax.numpy as jnp
from jax import lax
from jax.experimental import pallas as pl
from jax.experimental.pallas import tpu as pltpu
from jax.experimental.pallas import tpu_sc as plsc

NC = 2
NS = 16
NW = NC * NS
K = 128
LANES = 16


def _zero_vmem_2d(ref, rows, cols):
    def zrow(i, _):
        for f in range(cols // LANES):
            ref[i, pl.ds(f * LANES, LANES)] = jnp.zeros((LANES,), jnp.float32)
        return 0
    lax.fori_loop(0, rows, zrow, 0)


def _deg_body(col3, ew3, out, col_buf, ew_buf, stage, deg_sp):
    cid = lax.axis_index("c")
    sid = lax.axis_index("s")
    wid = sid * NC + cid
    C = col3.shape[1]
    seg = deg_sp.shape[0] // NS

    def zb(i, _):
        stage[pl.ds(i * LANES, LANES)] = jnp.zeros((LANES,), jnp.float32)
        return 0
    lax.fori_loop(0, seg // LANES, zb, 0)
    pltpu.sync_copy(stage, deg_sp.at[pl.ds(sid * seg, seg)])
    plsc.subcore_barrier()

    pltpu.sync_copy(col3.at[wid], col_buf)
    pltpu.sync_copy(ew3.at[wid], ew_buf)

    def body(i, _):
        pltpu.sync_copy(ew_buf.at[i], deg_sp.at[col_buf.at[i]], add=True)
        return 0
    lax.fori_loop(0, C, body, 0)
    plsc.subcore_barrier()
    pltpu.sync_copy(deg_sp.at[pl.ds(sid * seg, seg)],
                    out.at[pl.ds((cid * NS + sid) * seg, seg)])


def _core0_chunks(ck2):
    c0 = max(3, int(round(ck2 * _CORE0_FRAC / 3.0)) * 3)
    return min(c0, ck2 - 3)


_CORE0_FRAC = 0.4167

def _prop_body(hs, rc, ewp, out,
               ib0, ib1, ib2, eb0, eb1, eb2, gb0, gb1, gb2,
               sg0, sg1, sg2, ss0, ss1, ss2, acc_sp):
    cid = lax.axis_index("c")
    sid = lax.axis_index("s")
    CK2 = rc.shape[0] // NS
    Kc = gb0.shape[0]
    D = acc_sp.shape[1]
    rpt = acc_sp.shape[0] // NS

    ibs = (ib0, ib1, ib2)
    ebs = (eb0, eb1, eb2)
    gbs = (gb0, gb1, gb2)
    sgs = (sg0, sg1, sg2)
    sss = (ss0, ss1, ss2)

    def zrow(i, _):
        for f in range(D // LANES):
            gb0[i, pl.ds(f * LANES, LANES)] = jnp.zeros((LANES,), jnp.float32)
        return 0
    lax.fori_loop(0, Kc, zrow, 0)
    nfull = rpt // Kc
    rem = rpt - nfull * Kc

    def zc(t, _):
        pltpu.sync_copy(gb0.at[pl.ds(0, Kc)],
                        acc_sp.at[pl.ds(sid * rpt + t * Kc, Kc)])
        return 0
    lax.fori_loop(0, nfull, zc, 0)
    if rem:
        pltpu.sync_copy(gb0.at[pl.ds(0, rem)],
                        acc_sp.at[pl.ds(sid * rpt + nfull * Kc, rem)])
    plsc.subcore_barrier()

    def load_idx(b, r):
        pltpu.sync_copy(rc.at[r], ibs[b])
        pltpu.sync_copy(ewp.at[r], ebs[b])

    def fire_gather(b):
        pltpu.async_copy(hs.at[ibs[b].at[0]], gbs[b], sgs[b])

    def wait_gather(b):
        pltpu.make_async_copy(hs.at[ibs[b].at[0]], gbs[b], sgs[b]).wait()

    def fire_scatter(b):
        pltpu.async_copy(gbs[b], acc_sp.at[ibs[b].at[1]], sss[b], add=True)

    def wait_scatter(b):
        pltpu.make_async_copy(gbs[b], acc_sp.at[ibs[b].at[1]], sss[b]).wait()

    def mul(b):
        gb = gbs[b]
        eb = ebs[b]

        def mg(g, _):
            ewv = eb[0, pl.ds(g * LANES, LANES)]
            for j in range(LANES):
                s = ewv[j]
                e = g * LANES + j
                for f in range(D // LANES):
                    sl = pl.ds(f * LANES, LANES)
                    gb[e, sl] = gb[e, sl] * s
            return 0
        lax.fori_loop(0, Kc // LANES, mg, 0)

    def sub(r, b, bn, warm, last):
        wait_gather(b)
        if not last:
            if not warm:
                wait_scatter(bn)
            load_idx(bn, r + 1)
            fire_gather(bn)
        mul(b)
        fire_scatter(b)

    def pipeline(base, CKc):
        load_idx(0, base)
        fire_gather(0)
        sub(base, 0, 1, True, False)
        sub(base + 1, 1, 2, True, False)

        def main(t, _):
            r = base + 2 + t * 3
            sub(r, 2, 0, False, False)
            sub(r + 1, 0, 1, False, False)
            sub(r + 2, 1, 2, False, False)
            return 0
        lax.fori_loop(0, (CKc - 3) // 3, main, 0)
        sub(base + CKc - 1, 2, 0, False, True)
        wait_scatter(0)
        wait_scatter(1)
        wait_scatter(2)

    CK0 = _core0_chunks(CK2)
    CK1 = CK2 - CK0

    @pl.when(cid == 0)
    def _():
        pipeline(sid * CK0, CK0)

    @pl.when(cid == 1)
    def _():
        pipeline(NS * CK0 + sid * CK1, CK1)

    plsc.subcore_barrier()
    pltpu.sync_copy(acc_sp.at[pl.ds(sid * rpt, rpt)],
                    out.at[cid, pl.ds(sid * rpt, rpt)])


def _t1_body(x_ref, we_ref, w1_ref, degp_ref, hs1_ref, dis_ref):
    dp = degp_ref[...]
    deg = dp[0] + dp[1]
    d = jnp.where(deg > 0, lax.rsqrt(jnp.maximum(deg, 1e-12)), 0.0)
    h0 = jnp.maximum(
        jnp.dot(x_ref[...], we_ref[...], preferred_element_type=jnp.float32), 0.0)
    hs1_ref[...] = d * jnp.dot(h0, w1_ref[...],
                               preferred_element_type=jnp.float32)
    dis_ref[...] = d


def _t2_body(p_ref, dis_ref, b_ref, w3_ref, hs3_ref):
    pp = p_ref[...]
    d = dis_ref[...]
    h1 = jnp.maximum(d * (pp[0] + pp[1]) + b_ref[...], 0.0)
    hs3_ref[...] = d * jnp.dot(h1, w3_ref[...],
                               preferred_element_type=jnp.float32)


def _t3_body(p_ref, dis_ref, b_ref, out_ref):
    pp = p_ref[...]
    out_ref[...] = dis_ref[...] * (pp[0] + pp[1]) + b_ref[...]


def _divisor_le(n, cap):
    for d in range(min(cap, n), 0, -1):
        if n % d == 0:
            return d
    return 1


def kernel(x, edge_index, edge_weight, W_embed, W1, b1, W3, b3):
    N, D_in = x.shape
    D_h = W1.shape[0]
    D_out = W3.shape[1]
    E = edge_weight.shape[0]

    row = edge_index[0]
    col = edge_index[1]
    Kc = 112
    CT = NW * Kc
    CK = ((E + CT - 1) // CT + 2) // 3 * 3
    E_pad = CT * CK
    pad = E_pad - E
    ew = edge_weight
    if pad:
        row = jnp.concatenate([row, jnp.zeros((pad,), row.dtype)])
        col = jnp.concatenate([col, jnp.zeros((pad,), col.dtype)])
        ew = jnp.concatenate([ew, jnp.zeros((pad,), ew.dtype)])
    rc = jnp.stack([row.reshape(NW * CK, Kc),
                    col.reshape(NW * CK, Kc)], axis=1)
    ewp = ew.reshape(NW * CK, 1, Kc)
    col3 = col.reshape(NW, CK, Kc)
    ew3 = ew.reshape(NW, CK, Kc)

    mesh = plsc.VectorSubcoreMesh(core_axis_name="c", subcore_axis_name="s")

    seg = ((N + NS * 1024 - 1) // (NS * 1024)) * 1024
    deg_call = pl.kernel(
        _deg_body,
        out_type=jax.ShapeDtypeStruct((NC * NS * seg,), jnp.float32),
        mesh=mesh,
        scratch_types=[
            pltpu.VMEM((CK, Kc), jnp.int32),
            pltpu.VMEM((CK, Kc), jnp.float32),
            pltpu.VMEM((seg,), jnp.float32),
            pltpu.VMEM_SHARED((NS * seg,), jnp.float32),
        ],
    )
    deg_p = deg_call(col3, ew3)
    degp2 = deg_p.reshape(NC, NS * seg)[:, :N].reshape(NC, N, 1)

    rpt = (((N + NS - 1) // NS) + 7) // 8 * 8
    N_pad = NS * rpt
    prop_call = pl.kernel(
        _prop_body,
        out_type=jax.ShapeDtypeStruct((NC, N_pad, D_h), jnp.float32),
        mesh=mesh,
        scratch_types=(
            [pltpu.VMEM((2, Kc), jnp.int32) for _ in range(3)]
            + [pltpu.VMEM((1, Kc), jnp.float32) for _ in range(3)]
            + [pltpu.VMEM((Kc, D_h), jnp.float32) for _ in range(3)]
            + [pltpu.SemaphoreType.DMA for _ in range(6)]
            + [pltpu.VMEM_SHARED((N_pad, D_h), jnp.float32)]
        ),
    )

    B = _divisor_le(N, 1024)
    grid = (N // B,)
    t1 = pl.pallas_call(
        _t1_body,
        grid=grid,
        in_specs=[
            pl.BlockSpec((B, D_in), lambda r: (r, 0)),
            pl.BlockSpec((D_in, D_h), lambda r: (0, 0)),
            pl.BlockSpec((D_h, D_h), lambda r: (0, 0)),
            pl.BlockSpec((2, B, 1), lambda r: (0, r, 0)),
        ],
        out_specs=[
            pl.BlockSpec((B, D_h), lambda r: (r, 0)),
            pl.BlockSpec((B, 1), lambda r: (r, 0)),
        ],
        out_shape=[
            jax.ShapeDtypeStruct((N, D_h), jnp.float32),
            jax.ShapeDtypeStruct((N, 1), jnp.float32),
        ],
    )
    hs1, dis = t1(x, W_embed, W1, degp2)

    p1 = prop_call(hs1, rc, ewp)[:, :N]

    t2 = pl.pallas_call(
        _t2_body,
        grid=grid,
        in_specs=[
            pl.BlockSpec((NC, B, D_h), lambda r: (0, r, 0)),
            pl.BlockSpec((B, 1), lambda r: (r, 0)),
            pl.BlockSpec((1, D_h), lambda r: (0, 0)),
            pl.BlockSpec((D_h, D_out), lambda r: (0, 0)),
        ],
        out_specs=pl.BlockSpec((B, D_out), lambda r: (r, 0)),
        out_shape=jax.ShapeDtypeStruct((N, D_out), jnp.float32),
    )
    hs3 = t2(p1, dis, b1.reshape(1, D_h), W3)

    p3 = prop_call(hs3, rc, ewp)[:, :N]

    t3 = pl.pallas_call(
        _t3_body,
        grid=grid,
        in_specs=[
            pl.BlockSpec((NC, B, D_out), lambda r: (0, r, 0)),
            pl.BlockSpec((B, 1), lambda r: (r, 0)),
            pl.BlockSpec((1, D_out), lambda r: (0, 0)),
        ],
        out_specs=pl.BlockSpec((B, D_out), lambda r: (r, 0)),
        out_shape=jax.ShapeDtypeStruct((N, D_out), jnp.float32),
    )
    return t3(p3, dis, b3.reshape(1, D_out))


# speedup vs baseline: 1.2519x; 1.2519x over previous
import jax
import jax.numpy as jnp
from jax import lax
from jax.experimental import pallas as pl
from jax.experimental.pallas import tpu as pltpu
from jax.experimental.pallas import tpu_sc as plsc

NC = 2
NS = 16
NW = NC * NS
K = 128
LANES = 16


def _zero_vmem_2d(ref, rows, cols):
    def zrow(i, _):
        for f in range(cols // LANES):
            ref[i, pl.ds(f * LANES, LANES)] = jnp.zeros((LANES,), jnp.float32)
        return 0
    lax.fori_loop(0, rows, zrow, 0)


def _deg_body(col3, ew3, out, col_buf, ew_buf, stage, deg_sp):
    cid = lax.axis_index("c")
    sid = lax.axis_index("s")
    wid = sid * NC + cid
    C = col3.shape[1]
    seg = deg_sp.shape[0] // NS

    def zb(i, _):
        stage[pl.ds(i * LANES, LANES)] = jnp.zeros((LANES,), jnp.float32)
        return 0
    lax.fori_loop(0, seg // LANES, zb, 0)
    pltpu.sync_copy(stage, deg_sp.at[pl.ds(sid * seg, seg)])
    plsc.subcore_barrier()

    pltpu.sync_copy(col3.at[wid], col_buf)
    pltpu.sync_copy(ew3.at[wid], ew_buf)

    def body(i, _):
        pltpu.sync_copy(ew_buf.at[i], deg_sp.at[col_buf.at[i]], add=True)
        return 0
    lax.fori_loop(0, C, body, 0)
    plsc.subcore_barrier()
    pltpu.sync_copy(deg_sp.at[pl.ds(sid * seg, seg)],
                    out.at[pl.ds((cid * NS + sid) * seg, seg)])


def _core0_chunks(ck2):
    c0 = max(3, int(round(ck2 * _CORE0_FRAC / 3.0)) * 3)
    return min(c0, ck2 - 3)


_CORE0_FRAC = 0.5833

def _prop_body(hs, rc, ewp, out,
               ib0, ib1, ib2, eb0, eb1, eb2, gb0, gb1, gb2,
               sg0, sg1, sg2, ss0, ss1, ss2, acc_sp):
    cid = lax.axis_index("c")
    sid = lax.axis_index("s")
    CK2 = rc.shape[0] // NS
    Kc = gb0.shape[0]
    D = acc_sp.shape[1]
    rpt = acc_sp.shape[0] // NS

    ibs = (ib0, ib1, ib2)
    ebs = (eb0, eb1, eb2)
    gbs = (gb0, gb1, gb2)
    sgs = (sg0, sg1, sg2)
    sss = (ss0, ss1, ss2)

    def zrow(i, _):
        for f in range(D // LANES):
            gb0[i, pl.ds(f * LANES, LANES)] = jnp.zeros((LANES,), jnp.float32)
        return 0
    lax.fori_loop(0, Kc, zrow, 0)
    nfull = rpt // Kc
    rem = rpt - nfull * Kc

    def zc(t, _):
        pltpu.sync_copy(gb0.at[pl.ds(0, Kc)],
                        acc_sp.at[pl.ds(sid * rpt + t * Kc, Kc)])
        return 0
    lax.fori_loop(0, nfull, zc, 0)
    if rem:
        pltpu.sync_copy(gb0.at[pl.ds(0, rem)],
                        acc_sp.at[pl.ds(sid * rpt + nfull * Kc, rem)])
    plsc.subcore_barrier()

    def load_idx(b, r):
        pltpu.sync_copy(rc.at[r], ibs[b])
        pltpu.sync_copy(ewp.at[r], ebs[b])

    def fire_gather(b):
        pltpu.async_copy(hs.at[ibs[b].at[0]], gbs[b], sgs[b])

    def wait_gather(b):
        pltpu.make_async_copy(hs.at[ibs[b].at[0]], gbs[b], sgs[b]).wait()

    def fire_scatter(b):
        pltpu.async_copy(gbs[b], acc_sp.at[ibs[b].at[1]], sss[b], add=True)

    def wait_scatter(b):
        pltpu.make_async_copy(gbs[b], acc_sp.at[ibs[b].at[1]], sss[b]).wait()

    def mul(b):
        gb = gbs[b]
        eb = ebs[b]

        def mg(g, _):
            ewv = eb[0, pl.ds(g * LANES, LANES)]
            for j in range(LANES):
                s = ewv[j]
                e = g * LANES + j
                for f in range(D // LANES):
                    sl = pl.ds(f * LANES, LANES)
                    gb[e, sl] = gb[e, sl] * s
            return 0
        lax.fori_loop(0, Kc // LANES, mg, 0)

    def sub(r, b, bn, warm, last):
        wait_gather(b)
        if not last:
            if not warm:
                wait_scatter(bn)
            load_idx(bn, r + 1)
            fire_gather(bn)
        mul(b)
        fire_scatter(b)

    def pipeline(base, CKc):
        load_idx(0, base)
        fire_gather(0)
        sub(base, 0, 1, True, False)
        sub(base + 1, 1, 2, True, False)

        def main(t, _):
            r = base + 2 + t * 3
            sub(r, 2, 0, False, False)
            sub(r + 1, 0, 1, False, False)
            sub(r + 2, 1, 2, False, False)
            return 0
        lax.fori_loop(0, (CKc - 3) // 3, main, 0)
        sub(base + CKc - 1, 2, 0, False, True)
        wait_scatter(0)
        wait_scatter(1)
        wait_scatter(2)

    CK0 = _core0_chunks(CK2)
    CK1 = CK2 - CK0

    @pl.when(cid == 0)
    def _():
        pipeline(sid * CK0, CK0)

    @pl.when(cid == 1)
    def _():
        pipeline(NS * CK0 + sid * CK1, CK1)

    plsc.subcore_barrier()
    pltpu.sync_copy(acc_sp.at[pl.ds(sid * rpt, rpt)],
                    out.at[cid, pl.ds(sid * rpt, rpt)])


def _t1_body(x_ref, we_ref, w1_ref, degp_ref, hs1_ref, dis_ref):
    dp = degp_ref[...]
    deg = dp[0] + dp[1]
    d = jnp.where(deg > 0, lax.rsqrt(jnp.maximum(deg, 1e-12)), 0.0)
    h0 = jnp.maximum(
        jnp.dot(x_ref[...], we_ref[...], preferred_element_type=jnp.float32), 0.0)
    hs1_ref[...] = d * jnp.dot(h0, w1_ref[...],
                               preferred_element_type=jnp.float32)
    dis_ref[...] = d


def _t2_body(p_ref, dis_ref, b_ref, w3_ref, hs3_ref):
    pp = p_ref[...]
    d = dis_ref[...]
    h1 = jnp.maximum(d * (pp[0] + pp[1]) + b_ref[...], 0.0)
    hs3_ref[...] = d * jnp.dot(h1, w3_ref[...],
                               preferred_element_type=jnp.float32)


def _t3_body(p_ref, dis_ref, b_ref, out_ref):
    pp = p_ref[...]
    out_ref[...] = dis_ref[...] * (pp[0] + pp[1]) + b_ref[...]


def _divisor_le(n, cap):
    for d in range(min(cap, n), 0, -1):
        if n % d == 0:
            return d
    return 1


def kernel(x, edge_index, edge_weight, W_embed, W1, b1, W3, b3):
    N, D_in = x.shape
    D_h = W1.shape[0]
    D_out = W3.shape[1]
    E = edge_weight.shape[0]

    row = edge_index[0]
    col = edge_index[1]
    Kc = 112
    CT = NW * Kc
    CK = ((E + CT - 1) // CT + 2) // 3 * 3
    E_pad = CT * CK
    pad = E_pad - E
    ew = edge_weight
    if pad:
        row = jnp.concatenate([row, jnp.zeros((pad,), row.dtype)])
        col = jnp.concatenate([col, jnp.zeros((pad,), col.dtype)])
        ew = jnp.concatenate([ew, jnp.zeros((pad,), ew.dtype)])
    rc = jnp.stack([row.reshape(NW * CK, Kc),
                    col.reshape(NW * CK, Kc)], axis=1)
    ewp = ew.reshape(NW * CK, 1, Kc)
    col3 = col.reshape(NW, CK, Kc)
    ew3 = ew.reshape(NW, CK, Kc)

    mesh = plsc.VectorSubcoreMesh(core_axis_name="c", subcore_axis_name="s")

    seg = ((N + NS * 1024 - 1) // (NS * 1024)) * 1024
    deg_call = pl.kernel(
        _deg_body,
        out_type=jax.ShapeDtypeStruct((NC * NS * seg,), jnp.float32),
        mesh=mesh,
        scratch_types=[
            pltpu.VMEM((CK, Kc), jnp.int32),
            pltpu.VMEM((CK, Kc), jnp.float32),
            pltpu.VMEM((seg,), jnp.float32),
            pltpu.VMEM_SHARED((NS * seg,), jnp.float32),
        ],
    )
    deg_p = deg_call(col3, ew3)
    degp2 = deg_p.reshape(NC, NS * seg)[:, :N].reshape(NC, N, 1)

    rpt = (((N + NS - 1) // NS) + 7) // 8 * 8
    N_pad = NS * rpt
    prop_call = pl.kernel(
        _prop_body,
        out_type=jax.ShapeDtypeStruct((NC, N_pad, D_h), jnp.float32),
        mesh=mesh,
        scratch_types=(
            [pltpu.VMEM((2, Kc), jnp.int32) for _ in range(3)]
            + [pltpu.VMEM((1, Kc), jnp.float32) for _ in range(3)]
            + [pltpu.VMEM((Kc, D_h), jnp.float32) for _ in range(3)]
            + [pltpu.SemaphoreType.DMA for _ in range(6)]
            + [pltpu.VMEM_SHARED((N_pad, D_h), jnp.float32)]
        ),
    )

    B = _divisor_le(N, 1024)
    grid = (N // B,)
    t1 = pl.pallas_call(
        _t1_body,
        grid=grid,
        in_specs=[
            pl.BlockSpec((B, D_in), lambda r: (r, 0)),
            pl.BlockSpec((D_in, D_h), lambda r: (0, 0)),
            pl.BlockSpec((D_h, D_h), lambda r: (0, 0)),
            pl.BlockSpec((2, B, 1), lambda r: (0, r, 0)),
        ],
        out_specs=[
            pl.BlockSpec((B, D_h), lambda r: (r, 0)),
            pl.BlockSpec((B, 1), lambda r: (r, 0)),
        ],
        out_shape=[
            jax.ShapeDtypeStruct((N, D_h), jnp.float32),
            jax.ShapeDtypeStruct((N, 1), jnp.float32),
        ],
    )
    hs1, dis = t1(x, W_embed, W1, degp2)

    p1 = prop_call(hs1, rc, ewp)[:, :N]

    t2 = pl.pallas_call(
        _t2_body,
        grid=grid,
        in_specs=[
            pl.BlockSpec((NC, B, D_h), lambda r: (0, r, 0)),
            pl.BlockSpec((B, 1), lambda r: (r, 0)),
            pl.BlockSpec((1, D_h), lambda r: (0, 0)),
            pl.BlockSpec((D_h, D_out), lambda r: (0, 0)),
        ],
        out_specs=pl.BlockSpec((B, D_out), lambda r: (r, 0)),
        out_shape=jax.ShapeDtypeStruct((N, D_out), jnp.float32),
    )
    hs3 = t2(p1, dis, b1.reshape(1, D_h), W3)

    p3 = prop_call(hs3, rc, ewp)[:, :N]

    t3 = pl.pallas_call(
        _t3_body,
        grid=grid,
        in_specs=[
            pl.BlockSpec((NC, B, D_out), lambda r: (0, r, 0)),
            pl.BlockSpec((B, 1), lambda r: (r, 0)),
            pl.BlockSpec((1, D_out), lambda r: (0, 0)),
        ],
        out_specs=pl.BlockSpec((B, D_out), lambda r: (r, 0)),
        out_shape=jax.ShapeDtypeStruct((N, D_out), jnp.float32),
    )
    return t3(p3, dis, b3.reshape(1, D_out))


# speedup vs baseline: 1.2813x; 1.0235x over previous
import jax
import jax.numpy as jnp
from jax import lax
from jax.experimental import pallas as pl
from jax.experimental.pallas import tpu as pltpu
from jax.experimental.pallas import tpu_sc as plsc

NC = 2
NS = 16
NW = NC * NS
K = 128
LANES = 16


def _zero_vmem_2d(ref, rows, cols):
    def zrow(i, _):
        for f in range(cols // LANES):
            ref[i, pl.ds(f * LANES, LANES)] = jnp.zeros((LANES,), jnp.float32)
        return 0
    lax.fori_loop(0, rows, zrow, 0)


def _deg_body(col3, ew3, out, col_buf, ew_buf, stage, deg_sp):
    cid = lax.axis_index("c")
    sid = lax.axis_index("s")
    wid = sid * NC + cid
    C = col3.shape[1]
    seg = deg_sp.shape[0] // NS

    def zb(i, _):
        stage[pl.ds(i * LANES, LANES)] = jnp.zeros((LANES,), jnp.float32)
        return 0
    lax.fori_loop(0, seg // LANES, zb, 0)
    pltpu.sync_copy(stage, deg_sp.at[pl.ds(sid * seg, seg)])
    plsc.subcore_barrier()

    pltpu.sync_copy(col3.at[wid], col_buf)
    pltpu.sync_copy(ew3.at[wid], ew_buf)

    def body(i, _):
        pltpu.sync_copy(ew_buf.at[i], deg_sp.at[col_buf.at[i]], add=True)
        return 0
    lax.fori_loop(0, C, body, 0)
    plsc.subcore_barrier()
    pltpu.sync_copy(deg_sp.at[pl.ds(sid * seg, seg)],
                    out.at[pl.ds((cid * NS + sid) * seg, seg)])


def _core0_chunks(ck2):
    c0 = max(3, int(round(ck2 * _CORE0_FRAC / 3.0)) * 3)
    return min(c0, ck2 - 3)


_CORE0_FRAC = 0.5833

def _prop_body(hs, rce, out,
               ib0, ib1, ib2, gb0, gb1, gb2,
               sg0, sg1, sg2, ss0, ss1, ss2, acc_sp):
    cid = lax.axis_index("c")
    sid = lax.axis_index("s")
    CK2 = rce.shape[0] // NS
    Kc = gb0.shape[0]
    D = gb0.shape[1]
    rpt = acc_sp.shape[0] // NS

    ibs = (ib0, ib1, ib2)
    gbs = (gb0, gb1, gb2)
    sgs = (sg0, sg1, sg2)
    sss = (ss0, ss1, ss2)

    def zrow(i, _):
        for f in range(D // LANES):
            gb0[i, pl.ds(f * LANES, LANES)] = jnp.zeros((LANES,), jnp.float32)
        return 0
    lax.fori_loop(0, Kc, zrow, 0)
    nfull = rpt // Kc
    rem = rpt - nfull * Kc

    def zc(t, _):
        pltpu.sync_copy(gb0.at[pl.ds(0, Kc)],
                        acc_sp.at[pl.ds(sid * rpt + t * Kc, Kc)])
        return 0
    lax.fori_loop(0, nfull, zc, 0)
    if rem:
        pltpu.sync_copy(gb0.at[pl.ds(0, rem)],
                        acc_sp.at[pl.ds(sid * rpt + nfull * Kc, rem)])
    plsc.subcore_barrier()

    def load_idx(b, r):
        pltpu.sync_copy(rce.at[r], ibs[b])

    def fire_gather(b):
        pltpu.async_copy(hs.at[ibs[b].at[0]], gbs[b], sgs[b])

    def wait_gather(b):
        pltpu.make_async_copy(hs.at[ibs[b].at[0]], gbs[b], sgs[b]).wait()

    def fire_scatter(b):
        pltpu.async_copy(gbs[b], acc_sp.at[ibs[b].at[1]], sss[b], add=True)

    def wait_scatter(b):
        pltpu.make_async_copy(gbs[b], acc_sp.at[ibs[b].at[1]], sss[b]).wait()

    def mul(b):
        gb = gbs[b]
        ib = ibs[b]

        def mg(g, _):
            ewv = plsc.bitcast(ib[2, pl.ds(g * LANES, LANES)], jnp.float32)
            for j in range(LANES):
                s = ewv[j]
                e = g * LANES + j
                for f in range(D // LANES):
                    sl = pl.ds(f * LANES, LANES)
                    gb[e, sl] = gb[e, sl] * s
            return 0
        lax.fori_loop(0, Kc // LANES, mg, 0)

    def sub(r, b, bn, warm, last):
        wait_gather(b)
        if not last:
            if not warm:
                wait_scatter(bn)
            load_idx(bn, r + 1)
            fire_gather(bn)
        mul(b)
        fire_scatter(b)

    def pipeline(base, CKc):
        load_idx(0, base)
        fire_gather(0)
        sub(base, 0, 1, True, False)
        sub(base + 1, 1, 2, True, False)

        def main(t, _):
            r = base + 2 + t * 3
            sub(r, 2, 0, False, False)
            sub(r + 1, 0, 1, False, False)
            sub(r + 2, 1, 2, False, False)
            return 0
        lax.fori_loop(0, (CKc - 3) // 3, main, 0)
        sub(base + CKc - 1, 2, 0, False, True)
        wait_scatter(0)
        wait_scatter(1)
        wait_scatter(2)

    CK0 = _core0_chunks(CK2)
    CK1 = CK2 - CK0

    @pl.when(cid == 0)
    def _():
        pipeline(sid * CK0, CK0)

    @pl.when(cid == 1)
    def _():
        pipeline(NS * CK0 + sid * CK1, CK1)

    plsc.subcore_barrier()
    pltpu.sync_copy(acc_sp.at[pl.ds(sid * rpt, rpt)],
                    out.at[cid, pl.ds(sid * rpt, rpt)])


def _t1_body(x_ref, we_ref, w1_ref, degp_ref, hs1_ref, dis_ref):
    dp = degp_ref[...]
    deg = dp[0] + dp[1]
    d = jnp.where(deg > 0, lax.rsqrt(jnp.maximum(deg, 1e-12)), 0.0)
    h0 = jnp.maximum(
        jnp.dot(x_ref[...], we_ref[...], preferred_element_type=jnp.float32), 0.0)
    hs1_ref[...] = d * jnp.dot(h0, w1_ref[...],
                               preferred_element_type=jnp.float32)
    dis_ref[...] = d


def _t2_body(p_ref, dis_ref, b_ref, w3_ref, hs3_ref):
    pp = p_ref[...]
    d = dis_ref[...]
    h1 = jnp.maximum(d * (pp[0] + pp[1]) + b_ref[...], 0.0)
    hs3_ref[...] = d * jnp.dot(h1, w3_ref[...],
                               preferred_element_type=jnp.float32)


def _t3_body(p_ref, dis_ref, b_ref, out_ref):
    pp = p_ref[...]
    out_ref[...] = dis_ref[...] * (pp[0] + pp[1]) + b_ref[...]


def _divisor_le(n, cap):
    for d in range(min(cap, n), 0, -1):
        if n % d == 0:
            return d
    return 1


def kernel(x, edge_index, edge_weight, W_embed, W1, b1, W3, b3):
    N, D_in = x.shape
    D_h = W1.shape[0]
    D_out = W3.shape[1]
    E = edge_weight.shape[0]

    row = edge_index[0]
    col = edge_index[1]
    Kc = 112
    CT = NW * Kc
    CK = ((E + CT - 1) // CT + 2) // 3 * 3
    E_pad = CT * CK
    pad = E_pad - E
    ew = edge_weight
    if pad:
        row = jnp.concatenate([row, jnp.zeros((pad,), row.dtype)])
        col = jnp.concatenate([col, jnp.zeros((pad,), col.dtype)])
        ew = jnp.concatenate([ew, jnp.zeros((pad,), ew.dtype)])
    rce = jnp.stack([row.reshape(NW * CK, Kc),
                     col.reshape(NW * CK, Kc),
                     lax.bitcast_convert_type(ew, jnp.int32).reshape(NW * CK, Kc)],
                    axis=1)
    col3 = col.reshape(NW, CK, Kc)
    ew3 = ew.reshape(NW, CK, Kc)


    mesh = plsc.VectorSubcoreMesh(core_axis_name="c", subcore_axis_name="s")

    seg = ((N + NS * 1024 - 1) // (NS * 1024)) * 1024
    deg_call = pl.kernel(
        _deg_body,
        out_type=jax.ShapeDtypeStruct((NC * NS * seg,), jnp.float32),
        mesh=mesh,
        scratch_types=[
            pltpu.VMEM((CK, Kc), jnp.int32),
            pltpu.VMEM((CK, Kc), jnp.float32),
            pltpu.VMEM((seg,), jnp.float32),
            pltpu.VMEM_SHARED((NS * seg,), jnp.float32),
        ],
    )
    deg_p = deg_call(col3, ew3)
    degp2 = deg_p.reshape(NC, NS * seg)[:, :N].reshape(NC, N, 1)

    rpt = (((N + NS - 1) // NS) + 7) // 8 * 8
    N_pad = NS * rpt
    prop_call = pl.kernel(
        _prop_body,
        out_type=jax.ShapeDtypeStruct((NC, N_pad, D_h), jnp.float32),
        mesh=mesh,
        compiler_params=pltpu.CompilerParams(needs_layout_passes=False),
        scratch_types=(
            [pltpu.VMEM((3, Kc), jnp.int32) for _ in range(3)]
            + [pltpu.VMEM((Kc, D_h), jnp.float32) for _ in range(3)]
            + [pltpu.SemaphoreType.DMA for _ in range(6)]
            + [pltpu.VMEM_SHARED((N_pad, D_h), jnp.float32)]
        ),
    )

    B = _divisor_le(N // 16, 64) * 16
    grid = (N // B,)
    t1 = pl.pallas_call(
        _t1_body,
        grid=grid,
        in_specs=[
            pl.BlockSpec((B, D_in), lambda r: (r, 0)),
            pl.BlockSpec((D_in, D_h), lambda r: (0, 0)),
            pl.BlockSpec((D_h, D_h), lambda r: (0, 0)),
            pl.BlockSpec((2, B, 1), lambda r: (0, r, 0)),
        ],
        out_specs=[
            pl.BlockSpec((B, D_h), lambda r: (r, 0)),
            pl.BlockSpec((B, 1), lambda r: (r, 0)),
        ],
        out_shape=[
            jax.ShapeDtypeStruct((N, D_h), jnp.float32),
            jax.ShapeDtypeStruct((N, 1), jnp.float32),
        ],
    )
    hs1, dis = t1(x, W_embed, W1, degp2)

    p1 = prop_call(hs1, rce)[:, :N]

    t2 = pl.pallas_call(
        _t2_body,
        grid=grid,
        in_specs=[
            pl.BlockSpec((NC, B, D_h), lambda r: (0, r, 0)),
            pl.BlockSpec((B, 1), lambda r: (r, 0)),
            pl.BlockSpec((1, D_h), lambda r: (0, 0)),
            pl.BlockSpec((D_h, D_out), lambda r: (0, 0)),
        ],
        out_specs=pl.BlockSpec((B, D_out), lambda r: (r, 0)),
        out_shape=jax.ShapeDtypeStruct((N, D_out), jnp.float32),
    )
    hs3 = t2(p1, dis, b1.reshape(1, D_h), W3)

    p3 = prop_call(hs3, rce)[:, :N]

    t3 = pl.pallas_call(
        _t3_body,
        grid=grid,
        in_specs=[
            pl.BlockSpec((NC, B, D_out), lambda r: (0, r, 0)),
            pl.BlockSpec((B, 1), lambda r: (r, 0)),
            pl.BlockSpec((1, D_out), lambda r: (0, 0)),
        ],
        out_specs=pl.BlockSpec((B, D_out), lambda r: (r, 0)),
        out_shape=jax.ShapeDtypeStruct((N, D_out), jnp.float32),
    )
    return t3(p3, dis, b3.reshape(1, D_out))


# speedup vs baseline: 1.3337x; 1.0409x over previous
import jax
import jax.numpy as jnp
from jax import lax
from jax.experimental import pallas as pl
from jax.experimental.pallas import tpu as pltpu
from jax.experimental.pallas import tpu_sc as plsc

NC = 2
NS = 16
NW = NC * NS
LANES = 16


def _deg_body(col3, ew3, out, col_buf, ew_buf, stage, deg_sp):
    cid = lax.axis_index("c")
    sid = lax.axis_index("s")
    wid = sid * NC + cid
    C = col3.shape[1]
    seg = deg_sp.shape[0] // NS

    def zb(i, _):
        stage[pl.ds(i * LANES, LANES)] = jnp.zeros((LANES,), jnp.float32)
        return 0
    lax.fori_loop(0, seg // LANES, zb, 0)
    pltpu.sync_copy(stage, deg_sp.at[pl.ds(sid * seg, seg)])
    plsc.subcore_barrier()

    pltpu.sync_copy(col3.at[wid], col_buf)
    pltpu.sync_copy(ew3.at[wid], ew_buf)

    def body(i, _):
        pltpu.sync_copy(ew_buf.at[i], deg_sp.at[col_buf.at[i]], add=True)
        return 0
    lax.fori_loop(0, C, body, 0)
    plsc.subcore_barrier()
    pltpu.sync_copy(deg_sp.at[pl.ds(sid * seg, seg)],
                    out.at[pl.ds((cid * NS + sid) * seg, seg)])


def _core0_chunks(ck2):
    c0 = max(3, int(round(ck2 * _CORE0_FRAC / 3.0)) * 3)
    return min(c0, ck2 - 3)


_CORE0_FRAC = 0.60


def _prop_body(hs, rce, out,
               ib0, ib1, ib2, gb0, gb1, gb2,
               sg0, sg1, sg2, ss0, ss1, ss2, acc_sp):
    cid = lax.axis_index("c")
    sid = lax.axis_index("s")
    CK2 = rce.shape[0] // NS
    Kc = gb0.shape[0]
    D = gb0.shape[1]
    rpt = acc_sp.shape[0] // NS

    ibs = (ib0, ib1, ib2)
    gbs = (gb0, gb1, gb2)
    sgs = (sg0, sg1, sg2)
    sss = (ss0, ss1, ss2)

    def zrow(i, _):
        for f in range(D // LANES):
            gb0[i, pl.ds(f * LANES, LANES)] = jnp.zeros((LANES,), jnp.float32)
        return 0
    lax.fori_loop(0, Kc, zrow, 0)
    nfull = rpt // Kc
    rem = rpt - nfull * Kc

    def zc(t, _):
        pltpu.sync_copy(gb0.at[pl.ds(0, Kc)],
                        acc_sp.at[pl.ds(sid * rpt + t * Kc, Kc)])
        return 0
    lax.fori_loop(0, nfull, zc, 0)
    if rem:
        pltpu.sync_copy(gb0.at[pl.ds(0, rem)],
                        acc_sp.at[pl.ds(sid * rpt + nfull * Kc, rem)])
    plsc.subcore_barrier()

    def load_idx(b, r):
        pltpu.sync_copy(rce.at[r], ibs[b])

    def fire_gather(b):
        pltpu.async_copy(hs.at[ibs[b].at[0]], gbs[b], sgs[b])

    def wait_gather(b):
        pltpu.make_async_copy(hs.at[ibs[b].at[0]], gbs[b], sgs[b]).wait()

    def fire_scatter(b):
        pltpu.async_copy(gbs[b], acc_sp.at[ibs[b].at[1]], sss[b], add=True)

    def wait_scatter(b):
        pltpu.make_async_copy(gbs[b], acc_sp.at[ibs[b].at[1]], sss[b]).wait()

    def mul(b):
        gb = gbs[b]
        ib = ibs[b]

        def mg(g, _):
            ewv = plsc.bitcast(ib[2, pl.ds(g * LANES, LANES)], jnp.float32)
            for j in range(LANES):
                s = ewv[j]
                e = g * LANES + j
                for f in range(D // LANES):
                    sl = pl.ds(f * LANES, LANES)
                    gb[e, sl] = gb[e, sl] * s
            return 0
        lax.fori_loop(0, Kc // LANES, mg, 0)

    def sub(r, b, bn, warm, last):
        wait_gather(b)
        if not last:
            if not warm:
                wait_scatter(bn)
            load_idx(bn, r + 1)
            fire_gather(bn)
        mul(b)
        fire_scatter(b)

    def pipeline(base, CKc):
        load_idx(0, base)
        fire_gather(0)
        sub(base, 0, 1, True, False)
        sub(base + 1, 1, 2, True, False)

        def main(t, _):
            r = base + 2 + t * 3
            sub(r, 2, 0, False, False)
            sub(r + 1, 0, 1, False, False)
            sub(r + 2, 1, 2, False, False)
            return 0
        lax.fori_loop(0, (CKc - 3) // 3, main, 0)
        sub(base + CKc - 1, 2, 0, False, True)
        wait_scatter(0)
        wait_scatter(1)
        wait_scatter(2)

    CK0 = _core0_chunks(CK2)
    CK1 = CK2 - CK0

    @pl.when(cid == 0)
    def _():
        pipeline(sid * CK0, CK0)

    @pl.when(cid == 1)
    def _():
        pipeline(NS * CK0 + sid * CK1, CK1)

    plsc.subcore_barrier()
    pltpu.sync_copy(acc_sp.at[pl.ds(sid * rpt, rpt)],
                    out.at[cid, pl.ds(sid * rpt, rpt)])


def _t1_body(x_ref, we_ref, w1_ref, degp_ref, hs1_ref, dis_ref):
    dp = degp_ref[...]
    deg = dp[0] + dp[1]
    d = jnp.where(deg > 0, lax.rsqrt(jnp.maximum(deg, 1e-12)), 0.0)
    h0 = jnp.maximum(
        jnp.dot(x_ref[...], we_ref[...], preferred_element_type=jnp.float32), 0.0)
    hs1_ref[...] = d * jnp.dot(h0, w1_ref[...],
                               preferred_element_type=jnp.float32)
    dis_ref[...] = d


def _t2_body(p_ref, dis_ref, b_ref, w3_ref, hs3_ref):
    pp = p_ref[...]
    d = dis_ref[...]
    h1 = jnp.maximum(d * (pp[0] + pp[1]) + b_ref[...], 0.0)
    hs3_ref[...] = d * jnp.dot(h1, w3_ref[...],
                               preferred_element_type=jnp.float32)


def _t3_body(p_ref, dis_ref, b_ref, out_ref):
    pp = p_ref[...]
    out_ref[...] = dis_ref[...] * (pp[0] + pp[1]) + b_ref[...]


def _divisor_le(n, cap):
    for d in range(min(cap, n), 0, -1):
        if n % d == 0:
            return d
    return 1


def kernel(x, edge_index, edge_weight, W_embed, W1, b1, W3, b3):
    N, D_in = x.shape
    D_h = W1.shape[0]
    D_out = W3.shape[1]
    E = edge_weight.shape[0]

    row = edge_index[0]
    col = edge_index[1]
    Kc = 112
    CT = NW * Kc
    CK = ((E + CT - 1) // CT + 2) // 3 * 3
    E_pad = CT * CK
    pad = E_pad - E
    ew = edge_weight
    if pad:
        row = jnp.concatenate([row, jnp.zeros((pad,), row.dtype)])
        col = jnp.concatenate([col, jnp.zeros((pad,), col.dtype)])
        ew = jnp.concatenate([ew, jnp.zeros((pad,), ew.dtype)])
    rce = jnp.stack([row.reshape(NW * CK, Kc),
                     col.reshape(NW * CK, Kc),
                     lax.bitcast_convert_type(ew, jnp.int32).reshape(NW * CK, Kc)],
                    axis=1)
    col3 = col.reshape(NW, CK, Kc)
    ew3 = ew.reshape(NW, CK, Kc)

    mesh = plsc.VectorSubcoreMesh(core_axis_name="c", subcore_axis_name="s")

    seg = ((N + NS * 1024 - 1) // (NS * 1024)) * 1024
    deg_call = pl.kernel(
        _deg_body,
        out_type=jax.ShapeDtypeStruct((NC * NS * seg,), jnp.float32),
        mesh=mesh,
        scratch_types=[
            pltpu.VMEM((CK, Kc), jnp.int32),
            pltpu.VMEM((CK, Kc), jnp.float32),
            pltpu.VMEM((seg,), jnp.float32),
            pltpu.VMEM_SHARED((NS * seg,), jnp.float32),
        ],
    )
    deg_p = deg_call(col3, ew3)
    degp2 = deg_p.reshape(NC, NS * seg)[:, :N].reshape(NC, N, 1)

    rpt = (((N + NS - 1) // NS) + 7) // 8 * 8
    N_pad = NS * rpt
    prop_call = pl.kernel(
        _prop_body,
        out_type=jax.ShapeDtypeStruct((NC, N_pad, D_h), jnp.float32),
        mesh=mesh,
        compiler_params=pltpu.CompilerParams(needs_layout_passes=False),
        scratch_types=(
            [pltpu.VMEM((3, Kc), jnp.int32) for _ in range(3)]
            + [pltpu.VMEM((Kc, D_h), jnp.float32) for _ in range(3)]
            + [pltpu.SemaphoreType.DMA for _ in range(6)]
            + [pltpu.VMEM_SHARED((N_pad, D_h), jnp.float32)]
        ),
    )

    B = _divisor_le(N // 16, 64) * 16
    grid = (N // B,)
    t1 = pl.pallas_call(
        _t1_body,
        grid=grid,
        in_specs=[
            pl.BlockSpec((B, D_in), lambda r: (r, 0)),
            pl.BlockSpec((D_in, D_h), lambda r: (0, 0)),
            pl.BlockSpec((D_h, D_h), lambda r: (0, 0)),
            pl.BlockSpec((2, B, 1), lambda r: (0, r, 0)),
        ],
        out_specs=[
            pl.BlockSpec((B, D_h), lambda r: (r, 0)),
            pl.BlockSpec((B, 1), lambda r: (r, 0)),
        ],
        out_shape=[
            jax.ShapeDtypeStruct((N, D_h), jnp.float32),
            jax.ShapeDtypeStruct((N, 1), jnp.float32),
        ],
    )
    hs1, dis = t1(x, W_embed, W1, degp2)

    p1 = prop_call(hs1, rce)[:, :N]

    t2 = pl.pallas_call(
        _t2_body,
        grid=grid,
        in_specs=[
            pl.BlockSpec((NC, B, D_h), lambda r: (0, r, 0)),
            pl.BlockSpec((B, 1), lambda r: (r, 0)),
            pl.BlockSpec((1, D_h), lambda r: (0, 0)),
            pl.BlockSpec((D_h, D_out), lambda r: (0, 0)),
        ],
        out_specs=pl.BlockSpec((B, D_out), lambda r: (r, 0)),
        out_shape=jax.ShapeDtypeStruct((N, D_out), jnp.float32),
    )
    hs3 = t2(p1, dis, b1.reshape(1, D_h), W3)

    p3 = prop_call(hs3, rce)[:, :N]

    t3 = pl.pallas_call(
        _t3_body,
        grid=grid,
        in_specs=[
            pl.BlockSpec((NC, B, D_out), lambda r: (0, r, 0)),
            pl.BlockSpec((B, 1), lambda r: (r, 0)),
            pl.BlockSpec((1, D_out), lambda r: (0, 0)),
        ],
        out_specs=pl.BlockSpec((B, D_out), lambda r: (r, 0)),
        out_shape=jax.ShapeDtypeStruct((N, D_out), jnp.float32),
    )
    return t3(p3, dis, b3.reshape(1, D_out))
